# baseline (speedup 1.0000x reference)
import jax
import jax.numpy as jnp
import numpy as np
from jax.experimental import pallas as pl

_E = 64
_H = 4
_DH = _E // _H
_GRID = 40
_M = 65536
_B = 4
_N = 100000
_BN = 2000
_BM = 2048
_KKEEP = int(0.5 * _M)


def _embed_body(pn_ref, w_ref, b_ref, out_ref):
    out_ref[0] = jnp.dot(pn_ref[0], w_ref[...],
                         preferred_element_type=jnp.float32) + b_ref[...]


def _stats_body(feat_ref, mask_ref, sum_ref):
    mb = pl.program_id(1)

    @pl.when(mb == 0)
    def _():
        sum_ref[...] = jnp.zeros_like(sum_ref)

    f = feat_ref[0]
    m = mask_ref[0]
    sum_ref[0] += jnp.sum(f * m, axis=0, keepdims=True)


def _scores_body(feat_ref, wqe_ref, out_ref):
    out_ref[0] = jnp.dot(feat_ref[0], wqe_ref[0],
                         preferred_element_type=jnp.float32)


def _expsum_body(scores_ref, mask_ref, thr_ref, shift_ref, out_ref):
    mb = pl.program_id(1)

    @pl.when(mb == 0)
    def _():
        out_ref[...] = jnp.zeros_like(out_ref)

    s = scores_ref[0]
    m = mask_ref[0] > 0.0
    keep = (s >= thr_ref[0]) & m
    e = jnp.where(keep, jnp.exp(s - shift_ref[0]), 0.0)
    out_ref[0] += jnp.sum(e, axis=0, keepdims=True)


def _attn_out_body(feat_ref, scores_ref, mask_ref, thr_ref, shift_ref,
                   es_ref, wv_ref, wo_ref, rep_ref, attn_ref, out_ref):
    mb = pl.program_id(1)
    s = scores_ref[0]
    mcol = mask_ref[0]
    keep = (s >= thr_ref[0]) & (mcol > 0.0)
    e = jnp.where(keep, jnp.exp(s - shift_ref[0]), 0.0)
    es = es_ref[0]
    attn = jnp.where(es > 0.0, e / es, 1.0 / _M)
    attn_ref[0] = attn
    v = jnp.dot(feat_ref[0], wv_ref[...], preferred_element_type=jnp.float32)
    av = v * jnp.dot(attn, rep_ref[...], preferred_element_type=jnp.float32)
    of = jnp.dot(av, wo_ref[...], preferred_element_type=jnp.float32)
    of = jnp.where(mcol > 0.0, of, -1e9)
    pmax = jnp.max(of, axis=0, keepdims=True)

    @pl.when(mb == 0)
    def _():
        out_ref[...] = jnp.full_like(out_ref, -1e9)

    out_ref[0] = jnp.maximum(out_ref[0], pmax)


def kernel(x, W_embed, b_embed, Wq, Wk, Wv, Wo):
    B, N, _ = x.shape
    nb_n = N // _BN
    nb_m = _M // _BM

    coords = jnp.clip(jnp.floor(x * _GRID).astype(jnp.int32), 0, _GRID - 1)
    vid = coords[..., 0] * (_GRID * _GRID) + coords[..., 1] * _GRID + coords[..., 2]

    def _seg(pts_b, vid_b):
        ones = jnp.ones((pts_b.shape[0],), dtype=jnp.float32)
        counts = jax.ops.segment_sum(ones, vid_b, num_segments=_M)
        sums = jax.ops.segment_sum(pts_b, vid_b, num_segments=_M)
        return counts, sums

    counts, sums = jax.vmap(_seg)(x, vid)
    denom = jnp.maximum(counts, 1.0)[..., None]
    centroid = sums / denom
    norm_pts = x - jnp.take_along_axis(centroid, vid[..., None], axis=1)
    pn = jnp.concatenate([x, norm_pts], axis=-1)

    pt_feat = pl.pallas_call(
        _embed_body,
        grid=(B, nb_n),
        in_specs=[
            pl.BlockSpec((1, _BN, 6), lambda b, i: (b, i, 0)),
            pl.BlockSpec((6, _E), lambda b, i: (0, 0)),
            pl.BlockSpec((1, _E), lambda b, i: (0, 0)),
        ],
        out_specs=pl.BlockSpec((1, _BN, _E), lambda b, i: (b, i, 0)),
        out_shape=jax.ShapeDtypeStruct((B, N, _E), jnp.float32),
    )(pn, W_embed, b_embed.reshape(1, _E))

    vox_feat = jax.vmap(
        lambda f, v: jax.ops.segment_sum(f, v, num_segments=_M))(pt_feat, vid)
    vox_feat = vox_feat / denom
    mask_f = (counts > 0.0).astype(jnp.float32)[..., None]

    mask = counts > 0.0
    q = (vox_feat @ Wq).reshape(B, _M, _H, _DH)
    kk = (vox_feat @ Wk).reshape(B, _M, _H, _DH)
    mf = mask[:, :, None, None].astype(jnp.float32)
    nvalid = jnp.maximum(jnp.sum(mask.astype(jnp.float32), axis=1), 1.0)
    k_mean = jnp.sum(kk * mf, axis=1) / nvalid[:, None, None]
    scores = jnp.einsum('bmhd,bhd->bmh', q, k_mean) / float(np.sqrt(_DH))

    st = jnp.transpose(scores, (0, 2, 1))
    topv, _ = jax.lax.top_k(st, _KKEEP)
    thresh = topv[:, :, -1:].transpose(0, 2, 1)
    shift = topv[:, :, :1].transpose(0, 2, 1)

    expsum = pl.pallas_call(
        _expsum_body,
        grid=(B, nb_m),
        in_specs=[
            pl.BlockSpec((1, _BM, _H), lambda b, i: (b, i, 0)),
            pl.BlockSpec((1, _BM, 1), lambda b, i: (b, i, 0)),
            pl.BlockSpec((1, 1, _H), lambda b, i: (b, 0, 0)),
            pl.BlockSpec((1, 1, _H), lambda b, i: (b, 0, 0)),
        ],
        out_specs=pl.BlockSpec((1, 1, _H), lambda b, i: (b, 0, 0)),
        out_shape=jax.ShapeDtypeStruct((B, 1, _H), jnp.float32),
    )(scores, mask_f, thresh, shift)

    rep = (jnp.arange(_E)[None, :] // _DH == jnp.arange(_H)[:, None])
    rep = rep.astype(jnp.float32)

    attn, out = pl.pallas_call(
        _attn_out_body,
        grid=(B, nb_m),
        in_specs=[
            pl.BlockSpec((1, _BM, _E), lambda b, i: (b, i, 0)),
            pl.BlockSpec((1, _BM, _H), lambda b, i: (b, i, 0)),
            pl.BlockSpec((1, _BM, 1), lambda b, i: (b, i, 0)),
            pl.BlockSpec((1, 1, _H), lambda b, i: (b, 0, 0)),
            pl.BlockSpec((1, 1, _H), lambda b, i: (b, 0, 0)),
            pl.BlockSpec((1, 1, _H), lambda b, i: (b, 0, 0)),
            pl.BlockSpec((_E, _E), lambda b, i: (0, 0)),
            pl.BlockSpec((_E, _E), lambda b, i: (0, 0)),
            pl.BlockSpec((_H, _E), lambda b, i: (0, 0)),
        ],
        out_specs=[
            pl.BlockSpec((1, _BM, _H), lambda b, i: (b, i, 0)),
            pl.BlockSpec((1, 1, _E), lambda b, i: (b, 0, 0)),
        ],
        out_shape=[
            jax.ShapeDtypeStruct((B, _M, _H), jnp.float32),
            jax.ShapeDtypeStruct((B, 1, _E), jnp.float32),
        ],
    )(vox_feat, scores, mask_f, thresh, shift, expsum, Wv, Wo, rep)

    return out[:, 0, :], attn
